# baseline (speedup 1.0000x reference)
import functools

import jax
import jax.numpy as jnp
from jax import lax
from jax.experimental import pallas as pl
from jax.experimental.pallas import tpu as pltpu
from jax.experimental.pallas import tpu_sc as plsc

N_NODES = 10000
IN_DIM = 128
OUT_DIM = 128
N_EDGES = 320000

CHUNK = 128
NW = 32
CHUNKS_PER_W = 80
N_CHUNKS_P = NW * CHUNKS_PER_W
PAD_EDGES = N_CHUNKS_P * CHUNK - N_EDGES
GRP = 40
N_TRASH = 48
ACC_ROWS = N_NODES + N_TRASH

N_PIECES = N_NODES // CHUNK
TAIL_BASE = N_PIECES * CHUNK
ZTAIL_ROWS = ACC_ROWS - TAIL_BASE
DTAIL_ROWS = N_NODES - TAIL_BASE

_sc_mesh = plsc.VectorSubcoreMesh(core_axis_name="c", subcore_axis_name="s")


@functools.partial(
    pl.kernel,
    out_type=jax.ShapeDtypeStruct((2 * N_NODES, IN_DIM), jnp.float32),
    mesh=_sc_mesh,
    scratch_types=[
        pltpu.VMEM((GRP, CHUNK), jnp.int32),
        pltpu.VMEM((GRP, CHUNK), jnp.int32),
        pltpu.VMEM((CHUNK, IN_DIM), jnp.float32),
        pltpu.SemaphoreType.DMA,
        pltpu.SemaphoreType.DMA,
        pltpu.VMEM((CHUNK, IN_DIM), jnp.float32),
        pltpu.SemaphoreType.DMA,
        pltpu.SemaphoreType.DMA,
        pltpu.VMEM_SHARED((ACC_ROWS, IN_DIM), jnp.float32),
    ],
)
def _sc_aggregate(src_hbm, dst_hbm, x_hbm, out_hbm,
                  sgrp, dgrp, rowsA, gsemA, ssemA, rowsB, gsemB, ssemB, acc):
    core = lax.axis_index("c")
    sub = lax.axis_index("s")
    wid = core * 16 + sub

    zero16 = jnp.zeros((16,), jnp.float32)

    def zbody(i, carry):
        r = i // (IN_DIM // 16)
        j = i - r * (IN_DIM // 16)
        rowsA[r, pl.ds(j * 16, 16)] = zero16
        return carry

    lax.fori_loop(0, CHUNK * (IN_DIM // 16), zbody, 0)
    npiece = jnp.where(sub < 14, 5, 4)

    def zpiece(q, carry):
        pltpu.async_copy(rowsA, acc.at[pl.ds((sub + q * 16) * CHUNK, CHUNK)], gsemA)
        return carry

    lax.fori_loop(0, npiece, zpiece, 0)

    @pl.when(sub == 15)
    def _zero_tail():
        pltpu.async_copy(rowsA.at[pl.ds(0, ZTAIL_ROWS)],
                         acc.at[pl.ds(TAIL_BASE, ZTAIL_ROWS)], gsemB)

    def zdrain(q, carry):
        pltpu.make_async_copy(rowsA, acc.at[pl.ds(sub * CHUNK, CHUNK)], gsemA).wait()
        return carry

    lax.fori_loop(0, npiece, zdrain, 0)

    @pl.when(sub == 15)
    def _zero_tail_wait():
        pltpu.make_async_copy(rowsA.at[pl.ds(0, ZTAIL_ROWS)],
                              acc.at[pl.ds(TAIL_BASE, ZTAIL_ROWS)], gsemB).wait()

    plsc.subcore_barrier()

    def gather(i, rws, gsem):
        pltpu.async_copy(x_hbm.at[sgrp.at[i]], rws, gsem)

    def gwait(i, rws, gsem):
        pltpu.make_async_copy(x_hbm.at[sgrp.at[i]], rws, gsem).wait()

    def scatter(i, rws, ssem):
        pltpu.async_copy(rws, acc.at[dgrp.at[i]], ssem, add=True)

    def swait(i, rws, ssem):
        pltpu.make_async_copy(rws, acc.at[dgrp.at[i]], ssem).wait()

    for g in range(CHUNKS_PER_W // GRP):
        gbase = wid * CHUNKS_PER_W + g * GRP
        pltpu.sync_copy(src_hbm.at[pl.ds(gbase, GRP)], sgrp)
        pltpu.sync_copy(dst_hbm.at[pl.ds(gbase, GRP)], dgrp)

        gather(0, rowsA, gsemA)
        gather(1, rowsB, gsemB)
        gwait(0, rowsA, gsemA)
        scatter(0, rowsA, ssemA)

        def body(j, carry):
            k0 = 2 * j
            gwait(k0 + 1, rowsB, gsemB)
            scatter(k0 + 1, rowsB, ssemB)
            swait(k0, rowsA, ssemA)
            gather(k0 + 2, rowsA, gsemA)
            gwait(k0 + 2, rowsA, gsemA)
            scatter(k0 + 2, rowsA, ssemA)
            swait(k0 + 1, rowsB, ssemB)
            gather(k0 + 3, rowsB, gsemB)
            return carry

        lax.fori_loop(0, GRP // 2 - 1, body, 0)
        gwait(GRP - 1, rowsB, gsemB)
        scatter(GRP - 1, rowsB, ssemB)
        swait(GRP - 2, rowsA, ssemA)
        swait(GRP - 1, rowsB, ssemB)

    plsc.subcore_barrier()

    out0 = core * N_NODES

    for q in range(5):

        @pl.when(q < npiece)
        def _piece(q=q):
            base = (sub + q * 16) * CHUNK
            buf, wsem = (rowsA, gsemA) if q % 2 == 0 else (rowsB, gsemB)
            if q >= 2:
                pbase = (sub + (q - 2) * 16) * CHUNK
                pltpu.make_async_copy(
                    buf, out_hbm.at[pl.ds(out0 + pbase, CHUNK)], wsem).wait()
            pltpu.sync_copy(acc.at[pl.ds(base, CHUNK)], buf)
            pltpu.async_copy(buf, out_hbm.at[pl.ds(out0 + base, CHUNK)], wsem)

    for p in range(2, 5):

        @pl.when(jnp.logical_and(p >= npiece - 2, p < npiece))
        def _piece_wait(p=p):
            pbase = (sub + p * 16) * CHUNK
            buf, wsem = (rowsA, gsemA) if p % 2 == 0 else (rowsB, gsemB)
            pltpu.make_async_copy(
                buf, out_hbm.at[pl.ds(out0 + pbase, CHUNK)], wsem).wait()

    @pl.when(sub == 15)
    def _drain_tail():
        pltpu.sync_copy(acc.at[pl.ds(TAIL_BASE, DTAIL_ROWS)],
                        rowsA.at[pl.ds(0, DTAIL_ROWS)])
        pltpu.sync_copy(rowsA.at[pl.ds(0, DTAIL_ROWS)],
                        out_hbm.at[pl.ds(out0 + TAIL_BASE, DTAIL_ROWS)])


_M_BLK = 2000


def _tc_body(x_ref, a0_ref, a1_ref, w_ref, b_ref, o_ref):
    h = x_ref[...] + a0_ref[...] + a1_ref[...]
    o_ref[...] = (
        jnp.dot(h, w_ref[...], preferred_element_type=jnp.float32) + b_ref[...]
    )


def _tc_mlp(x, agg2, W, b2):
    n_blk = N_NODES // _M_BLK
    return pl.pallas_call(
        _tc_body,
        grid=(n_blk,),
        in_specs=[
            pl.BlockSpec((_M_BLK, IN_DIM), lambda i: (i, 0)),
            pl.BlockSpec((_M_BLK, IN_DIM), lambda i: (i, 0)),
            pl.BlockSpec((_M_BLK, IN_DIM), lambda i: (i + n_blk, 0)),
            pl.BlockSpec((IN_DIM, OUT_DIM), lambda i: (0, 0)),
            pl.BlockSpec((1, OUT_DIM), lambda i: (0, 0)),
        ],
        out_specs=pl.BlockSpec((_M_BLK, OUT_DIM), lambda i: (i, 0)),
        out_shape=jax.ShapeDtypeStruct((N_NODES, OUT_DIM), jnp.float32),
    )(x, agg2, agg2, W, b2)


def kernel(x, edge_index, W, b):
    ei = edge_index.astype(jnp.int32)
    pad_iota = jnp.arange(PAD_EDGES, dtype=jnp.int32)
    pad_src = (pad_iota * 131) % N_NODES
    pad_dst = N_NODES + pad_iota % N_TRASH
    src2 = jnp.concatenate([ei[0], pad_src]).reshape(N_CHUNKS_P, CHUNK)
    dst2 = jnp.concatenate([ei[1], pad_dst]).reshape(N_CHUNKS_P, CHUNK)
    agg2 = _sc_aggregate(src2, dst2, x)
    return _tc_mlp(x, agg2, W, b.reshape(1, OUT_DIM))


# speedup vs baseline: 1.1521x; 1.1521x over previous
import functools

import jax
import jax.numpy as jnp
from jax import lax
from jax.experimental import pallas as pl
from jax.experimental.pallas import tpu as pltpu
from jax.experimental.pallas import tpu_sc as plsc

N_NODES = 10000
IN_DIM = 128
OUT_DIM = 128
N_EDGES = 320000

CHUNK = 128
NW = 32
CHUNKS_PER_W = 80
N_CHUNKS_P = NW * CHUNKS_PER_W
PAD_EDGES = N_CHUNKS_P * CHUNK - N_EDGES
GRP = 40
N_TRASH = 48
ACC_ROWS = N_NODES + N_TRASH

N_PIECES = N_NODES // CHUNK
TAIL_BASE = N_PIECES * CHUNK
ZTAIL_ROWS = ACC_ROWS - TAIL_BASE
DTAIL_ROWS = N_NODES - TAIL_BASE

_sc_mesh = plsc.VectorSubcoreMesh(core_axis_name="c", subcore_axis_name="s")


@functools.partial(
    pl.kernel,
    out_type=jax.ShapeDtypeStruct((2 * N_NODES, IN_DIM), jnp.float32),
    mesh=_sc_mesh,
    scratch_types=[
        pltpu.VMEM((GRP, CHUNK), jnp.int32),
        pltpu.VMEM((GRP, CHUNK), jnp.int32),
        pltpu.VMEM((CHUNK, IN_DIM), jnp.float32),
        pltpu.SemaphoreType.DMA,
        pltpu.SemaphoreType.DMA,
        pltpu.VMEM((CHUNK, IN_DIM), jnp.float32),
        pltpu.SemaphoreType.DMA,
        pltpu.SemaphoreType.DMA,
        pltpu.VMEM_SHARED((ACC_ROWS, IN_DIM), jnp.float32),
    ],
)
def _sc_aggregate(src_hbm, dst_hbm, x_hbm, out_hbm,
                  sgrp, dgrp, rowsA, gsemA, ssemA, rowsB, gsemB, ssemB, acc):
    core = lax.axis_index("c")
    sub = lax.axis_index("s")
    wid = core * 16 + sub

    zero16 = jnp.zeros((16,), jnp.float32)

    def zbody(i, carry):
        r = i // (IN_DIM // 16)
        j = i - r * (IN_DIM // 16)
        rowsA[r, pl.ds(j * 16, 16)] = zero16
        return carry

    lax.fori_loop(0, CHUNK * (IN_DIM // 16), zbody, 0)
    npiece = jnp.where(sub < 14, 5, 4)

    def zpiece(q, carry):
        pltpu.async_copy(rowsA, acc.at[pl.ds((sub + q * 16) * CHUNK, CHUNK)], gsemA)
        return carry

    lax.fori_loop(0, npiece, zpiece, 0)

    @pl.when(sub == 15)
    def _zero_tail():
        pltpu.async_copy(rowsA.at[pl.ds(0, ZTAIL_ROWS)],
                         acc.at[pl.ds(TAIL_BASE, ZTAIL_ROWS)], gsemB)

    def zdrain(q, carry):
        pltpu.make_async_copy(rowsA, acc.at[pl.ds(sub * CHUNK, CHUNK)], gsemA).wait()
        return carry

    lax.fori_loop(0, npiece, zdrain, 0)

    @pl.when(sub == 15)
    def _zero_tail_wait():
        pltpu.make_async_copy(rowsA.at[pl.ds(0, ZTAIL_ROWS)],
                              acc.at[pl.ds(TAIL_BASE, ZTAIL_ROWS)], gsemB).wait()

    plsc.subcore_barrier()

    def gather(i, rws, gsem):
        pltpu.async_copy(x_hbm.at[sgrp.at[i]], rws, gsem)

    def gwait(i, rws, gsem):
        pltpu.make_async_copy(x_hbm.at[sgrp.at[i]], rws, gsem).wait()

    for g in range(CHUNKS_PER_W // GRP):
        gbase = wid * CHUNKS_PER_W + g * GRP
        pltpu.sync_copy(src_hbm.at[pl.ds(gbase, GRP)], sgrp)
        pltpu.sync_copy(dst_hbm.at[pl.ds(gbase, GRP)], dgrp)

        gather(0, rowsA, gsemA)
        gather(1, rowsB, gsemB)

        def body(j, carry):
            k0 = 2 * j
            gwait(k0, rowsA, gsemA)
            pltpu.sync_copy(rowsA, acc.at[dgrp.at[k0]], add=True)
            gather(k0 + 2, rowsA, gsemA)
            gwait(k0 + 1, rowsB, gsemB)
            pltpu.sync_copy(rowsB, acc.at[dgrp.at[k0 + 1]], add=True)
            gather(k0 + 3, rowsB, gsemB)
            return carry

        lax.fori_loop(0, GRP // 2 - 1, body, 0)
        gwait(GRP - 2, rowsA, gsemA)
        pltpu.sync_copy(rowsA, acc.at[dgrp.at[GRP - 2]], add=True)
        gwait(GRP - 1, rowsB, gsemB)
        pltpu.sync_copy(rowsB, acc.at[dgrp.at[GRP - 1]], add=True)

    plsc.subcore_barrier()

    out0 = core * N_NODES

    for q in range(5):

        @pl.when(q < npiece)
        def _piece(q=q):
            base = (sub + q * 16) * CHUNK
            buf, wsem = (rowsA, gsemA) if q % 2 == 0 else (rowsB, gsemB)
            if q >= 2:
                pbase = (sub + (q - 2) * 16) * CHUNK
                pltpu.make_async_copy(
                    buf, out_hbm.at[pl.ds(out0 + pbase, CHUNK)], wsem).wait()
            pltpu.sync_copy(acc.at[pl.ds(base, CHUNK)], buf)
            pltpu.async_copy(buf, out_hbm.at[pl.ds(out0 + base, CHUNK)], wsem)

    for p in range(2, 5):

        @pl.when(jnp.logical_and(p >= npiece - 2, p < npiece))
        def _piece_wait(p=p):
            pbase = (sub + p * 16) * CHUNK
            buf, wsem = (rowsA, gsemA) if p % 2 == 0 else (rowsB, gsemB)
            pltpu.make_async_copy(
                buf, out_hbm.at[pl.ds(out0 + pbase, CHUNK)], wsem).wait()

    @pl.when(sub == 15)
    def _drain_tail():
        pltpu.sync_copy(acc.at[pl.ds(TAIL_BASE, DTAIL_ROWS)],
                        rowsA.at[pl.ds(0, DTAIL_ROWS)])
        pltpu.sync_copy(rowsA.at[pl.ds(0, DTAIL_ROWS)],
                        out_hbm.at[pl.ds(out0 + TAIL_BASE, DTAIL_ROWS)])


_M_BLK = 2000


def _tc_body(x_ref, a0_ref, a1_ref, w_ref, b_ref, o_ref):
    h = x_ref[...] + a0_ref[...] + a1_ref[...]
    o_ref[...] = (
        jnp.dot(h, w_ref[...], preferred_element_type=jnp.float32) + b_ref[...]
    )


def _tc_mlp(x, agg2, W, b2):
    n_blk = N_NODES // _M_BLK
    return pl.pallas_call(
        _tc_body,
        grid=(n_blk,),
        in_specs=[
            pl.BlockSpec((_M_BLK, IN_DIM), lambda i: (i, 0)),
            pl.BlockSpec((_M_BLK, IN_DIM), lambda i: (i, 0)),
            pl.BlockSpec((_M_BLK, IN_DIM), lambda i: (i + n_blk, 0)),
            pl.BlockSpec((IN_DIM, OUT_DIM), lambda i: (0, 0)),
            pl.BlockSpec((1, OUT_DIM), lambda i: (0, 0)),
        ],
        out_specs=pl.BlockSpec((_M_BLK, OUT_DIM), lambda i: (i, 0)),
        out_shape=jax.ShapeDtypeStruct((N_NODES, OUT_DIM), jnp.float32),
    )(x, agg2, agg2, W, b2)


def kernel(x, edge_index, W, b):
    ei = edge_index.astype(jnp.int32)
    pad_iota = jnp.arange(PAD_EDGES, dtype=jnp.int32)
    pad_src = (pad_iota * 131) % N_NODES
    pad_dst = N_NODES + pad_iota % N_TRASH
    src2 = jnp.concatenate([ei[0], pad_src]).reshape(N_CHUNKS_P, CHUNK)
    dst2 = jnp.concatenate([ei[1], pad_dst]).reshape(N_CHUNKS_P, CHUNK)
    agg2 = _sc_aggregate(src2, dst2, x)
    return _tc_mlp(x, agg2, W, b.reshape(1, OUT_DIM))


# speedup vs baseline: 1.2076x; 1.0482x over previous
import functools

import jax
import jax.numpy as jnp
from jax import lax
from jax.experimental import pallas as pl
from jax.experimental.pallas import tpu as pltpu
from jax.experimental.pallas import tpu_sc as plsc

N_NODES = 10000
IN_DIM = 128
OUT_DIM = 128
N_EDGES = 320000

CHUNK = 128
NW = 32
NPW = 81
N_CHUNKS_P = NW * NPW
PAD_EDGES = N_CHUNKS_P * CHUNK - N_EDGES
N_TRASH = 48
ACC_ROWS = N_NODES + N_TRASH
VPC = IN_DIM // 16

N_PIECES = N_NODES // CHUNK
TAIL_BASE = N_PIECES * CHUNK
ZTAIL_ROWS = ACC_ROWS - TAIL_BASE
DTAIL_ROWS = N_NODES - TAIL_BASE

_sc_mesh = plsc.VectorSubcoreMesh(core_axis_name="c", subcore_axis_name="s")


@functools.partial(
    pl.kernel,
    out_type=jax.ShapeDtypeStruct((2 * N_NODES, IN_DIM), jnp.float32),
    mesh=_sc_mesh,
    scratch_types=[
        pltpu.VMEM((3, CHUNK), jnp.int32),
        pltpu.VMEM((3, CHUNK), jnp.int32),
        pltpu.VMEM((3, CHUNK), jnp.int32),
        pltpu.VMEM((3, CHUNK, IN_DIM), jnp.float32),
        pltpu.SemaphoreType.DMA,
        pltpu.SemaphoreType.DMA,
        pltpu.SemaphoreType.DMA,
        pltpu.SemaphoreType.DMA,
        pltpu.SemaphoreType.DMA,
        pltpu.SemaphoreType.DMA,
        pltpu.SemaphoreType.DMA,
        pltpu.SemaphoreType.DMA,
        pltpu.SemaphoreType.DMA,
        pltpu.VMEM_SHARED((ACC_ROWS, IN_DIM), jnp.float32),
    ],
)
def _sc_aggregate(pk_hbm, x_hbm, out_hbm, pbuf, sbuf, dbuf, rows,
                  psem0, psem1, psem2, gsem0, gsem1, gsem2,
                  ssem0, ssem1, ssem2, acc):
    psem = (psem0, psem1, psem2)
    gsem = (gsem0, gsem1, gsem2)
    ssem = (ssem0, ssem1, ssem2)
    core = lax.axis_index("c")
    sub = lax.axis_index("s")
    wid = core * 16 + sub
    base0 = wid * NPW

    zero16 = jnp.zeros((16,), jnp.float32)

    def zbody(i, carry):
        r = i // VPC
        j = i - r * VPC
        rows[0, r, pl.ds(j * 16, 16)] = zero16
        return carry

    lax.fori_loop(0, CHUNK * VPC, zbody, 0)
    zsrc = rows.at[0]
    npiece = jnp.where(sub < 14, 5, 4)

    def zpiece(q, carry):
        pltpu.async_copy(zsrc, acc.at[pl.ds((sub + q * 16) * CHUNK, CHUNK)], gsem0)
        return carry

    lax.fori_loop(0, npiece, zpiece, 0)

    @pl.when(sub == 15)
    def _zero_tail():
        pltpu.async_copy(zsrc.at[pl.ds(0, ZTAIL_ROWS)],
                         acc.at[pl.ds(TAIL_BASE, ZTAIL_ROWS)], gsem1)

    def zdrain(q, carry):
        pltpu.make_async_copy(zsrc, acc.at[pl.ds(sub * CHUNK, CHUNK)], gsem0).wait()
        return carry

    lax.fori_loop(0, npiece, zdrain, 0)

    @pl.when(sub == 15)
    def _zero_tail_wait():
        pltpu.make_async_copy(zsrc.at[pl.ds(0, ZTAIL_ROWS)],
                              acc.at[pl.ds(TAIL_BASE, ZTAIL_ROWS)], gsem1).wait()

    plsc.subcore_barrier()

    def P(k, m):
        pltpu.async_copy(pk_hbm.at[pl.ds((base0 + k) * CHUNK, CHUNK)],
                         pbuf.at[m], psem[m])

    def U(k, m):
        pltpu.make_async_copy(pk_hbm.at[pl.ds((base0 + k) * CHUNK, CHUNK)],
                              pbuf.at[m], psem[m]).wait()
        for j in range(VPC):
            p = pbuf[m, pl.ds(j * 16, 16)]
            sbuf[m, pl.ds(j * 16, 16)] = p & 16383
            dbuf[m, pl.ds(j * 16, 16)] = p >> 14

    def G(m):
        pltpu.async_copy(x_hbm.at[sbuf.at[m]], rows.at[m], gsem[m])

    def GW(m):
        pltpu.make_async_copy(x_hbm.at[sbuf.at[m]], rows.at[m], gsem[m]).wait()

    def S(m):
        pltpu.async_copy(rows.at[m], acc.at[dbuf.at[m]], ssem[m], add=True)

    def SW(m):
        pltpu.make_async_copy(rows.at[m], acc.at[dbuf.at[m]], ssem[m]).wait()

    P(0, 0)
    P(1, 1)
    P(2, 2)
    U(0, 0)
    P(3, 0)
    G(0)
    U(1, 1)
    P(4, 1)
    G(1)
    GW(0)
    S(0)
    U(2, 2)
    P(5, 2)
    G(2)

    def body(t, carry):
        k = 3 * t
        for off, m, m2 in ((1, 1, 0), (2, 2, 1), (3, 0, 2)):
            GW(m)
            S(m)
            SW(m2)
            U(k + off + 2, m2)
            P(k + off + 5, m2)
            G(m2)
        return carry

    lax.fori_loop(0, 25, body, 0)

    for k in range(76, 81):
        m = k % 3
        m2 = (k + 2) % 3
        GW(m)
        S(m)
        SW(m2)
        if k + 2 <= 80:
            U(k + 2, m2)
            G(m2)
    SW(80 % 3)

    plsc.subcore_barrier()

    out0 = core * N_NODES

    for q in range(5):

        @pl.when(q < npiece)
        def _piece(q=q):
            base = (sub + q * 16) * CHUNK
            buf, wsem = (rows.at[0], gsem0) if q % 2 == 0 else (rows.at[1], gsem1)
            if q >= 2:
                pbase = (sub + (q - 2) * 16) * CHUNK
                pltpu.make_async_copy(
                    buf, out_hbm.at[pl.ds(out0 + pbase, CHUNK)], wsem).wait()
            pltpu.sync_copy(acc.at[pl.ds(base, CHUNK)], buf)
            pltpu.async_copy(buf, out_hbm.at[pl.ds(out0 + base, CHUNK)], wsem)

    for p in range(2, 5):

        @pl.when(jnp.logical_and(p >= npiece - 2, p < npiece))
        def _piece_wait(p=p):
            pbase = (sub + p * 16) * CHUNK
            buf, wsem = (rows.at[0], gsem0) if p % 2 == 0 else (rows.at[1], gsem1)
            pltpu.make_async_copy(
                buf, out_hbm.at[pl.ds(out0 + pbase, CHUNK)], wsem).wait()

    @pl.when(sub == 15)
    def _drain_tail():
        pltpu.sync_copy(acc.at[pl.ds(TAIL_BASE, DTAIL_ROWS)],
                        rows.at[0].at[pl.ds(0, DTAIL_ROWS)])
        pltpu.sync_copy(rows.at[0].at[pl.ds(0, DTAIL_ROWS)],
                        out_hbm.at[pl.ds(out0 + TAIL_BASE, DTAIL_ROWS)])


_M_BLK = 2000


def _tc_body(x_ref, a0_ref, a1_ref, w_ref, b_ref, o_ref):
    h = x_ref[...] + a0_ref[...] + a1_ref[...]
    o_ref[...] = (
        jnp.dot(h, w_ref[...], preferred_element_type=jnp.float32) + b_ref[...]
    )


def _tc_mlp(x, agg2, W, b2):
    n_blk = N_NODES // _M_BLK
    return pl.pallas_call(
        _tc_body,
        grid=(n_blk,),
        in_specs=[
            pl.BlockSpec((_M_BLK, IN_DIM), lambda i: (i, 0)),
            pl.BlockSpec((_M_BLK, IN_DIM), lambda i: (i, 0)),
            pl.BlockSpec((_M_BLK, IN_DIM), lambda i: (i + n_blk, 0)),
            pl.BlockSpec((IN_DIM, OUT_DIM), lambda i: (0, 0)),
            pl.BlockSpec((1, OUT_DIM), lambda i: (0, 0)),
        ],
        out_specs=pl.BlockSpec((_M_BLK, OUT_DIM), lambda i: (i, 0)),
        out_shape=jax.ShapeDtypeStruct((N_NODES, OUT_DIM), jnp.float32),
    )(x, agg2, agg2, W, b2)


def kernel(x, edge_index, W, b):
    ei = edge_index.astype(jnp.int32)
    pad_iota = jnp.arange(PAD_EDGES, dtype=jnp.int32)
    pad_src = (pad_iota * 131) % N_NODES
    pad_dst = N_NODES + pad_iota % N_TRASH
    src_p = jnp.concatenate([ei[0], pad_src])
    dst_p = jnp.concatenate([ei[1], pad_dst])
    packed = dst_p * 16384 + src_p
    agg2 = _sc_aggregate(packed, x)
    return _tc_mlp(x, agg2, W, b.reshape(1, OUT_DIM))
